# baseline (speedup 1.0000x reference)
import functools

import jax
import jax.numpy as jnp
from jax import lax
from jax.experimental import pallas as pl
from jax.experimental.pallas import tpu as pltpu
from jax.experimental.pallas import tpu_sc as plsc

V = 1000
D = 1000
DPAD = 1024
B = 1024 * 50
NC, NS = 2, 16
NW = NC * NS
NTC = B // 128
KCH = DPAD // 128
TAIL = D - 896


def _sc_gather_t(table_r, idx):
    mesh = plsc.VectorSubcoreMesh(core_axis_name="c", subcore_axis_name="s")

    @functools.partial(
        pl.kernel,
        mesh=mesh,
        compiler_params=pltpu.CompilerParams(needs_layout_passes=False),
        out_type=jax.ShapeDtypeStruct((D, B), jnp.float32),
        scratch_types=[
            pltpu.VMEM((128,), jnp.int32),
            pltpu.VMEM((128,), jnp.int32),
            pltpu.VMEM((128,), jnp.int32),
            pltpu.VMEM((128,), jnp.int32),
            pltpu.VMEM((128, 128), jnp.float32),
            pltpu.VMEM((128, 128), jnp.float32),
            pltpu.VMEM((128, 128), jnp.float32),
            pltpu.VMEM((128, 128), jnp.float32),
            pltpu.VMEM((128, 128), jnp.float32),
            pltpu.SemaphoreType.DMA,
            pltpu.SemaphoreType.DMA,
            pltpu.SemaphoreType.DMA,
            pltpu.SemaphoreType.DMA,
            pltpu.SemaphoreType.DMA,
        ],
    )
    def k(table_hbm, idx_hbm, out_hbm, idx_v, idx8_v, gidx0, gidx1,
          rows0, rows1, blk0, blk1, blk7,
          sem_g0, sem_g1, sem_w0, sem_w1, sem_w7):
        cid = lax.axis_index("c")
        sid = lax.axis_index("s")
        wid = sid * NC + cid
        ntc = jnp.where(wid < NTC % NW, NTC // NW + 1, NTC // NW)

        rows = (rows0, rows1)
        gidx = (gidx0, gidx1)
        sem_g = (sem_g0, sem_g1)
        blks = (blk0, blk1)
        sem_w = (sem_w0, sem_w1)

        iota16 = lax.broadcasted_iota(jnp.int32, (16,), 0)
        iotas = [iota16 + 16 * sg for sg in range(8)]
        SKEW = 8
        skew16 = iota16 * SKEW

        def set_gidx(p, kk):
            for s in range(8):
                gidx[p][pl.ds(16 * s, 16)] = idx8_v[pl.ds(16 * s, 16)] + kk

        def gather_start(p):
            pltpu.async_copy(table_hbm.at[gidx[p]], rows[p], sem_g[p])

        def gather_wait(p):
            pltpu.make_async_copy(table_hbm.at[gidx[p]], rows[p], sem_g[p]).wait()

        def transpose(src, dst, limit):
            def body(j, carry):
                for dd in range(8):
                    d = j * 8 + dd
                    dvec = (skew16 + d) & 127
                    mask = dvec < limit if limit < 128 else None
                    for sg in range(8):
                        vals = plsc.load_gather(src, [iotas[sg], dvec])
                        plsc.store_scatter(dst, [dvec, iotas[sg]], vals, mask=mask)
                return carry

            lax.fori_loop(0, 16, body, 0)

        def write_start(kk, t, p):
            pltpu.async_copy(
                blks[p],
                out_hbm.at[pl.ds(kk * 128, 128), pl.ds(t * 128, 128)],
                sem_w[p],
            )

        def write_wait(kk, t, p):
            pltpu.make_async_copy(
                blks[p],
                out_hbm.at[pl.ds(kk * 128, 128), pl.ds(t * 128, 128)],
                sem_w[p],
            ).wait()

        def tile_col(ti, carry):
            t = wid + ti * NW
            pltpu.sync_copy(idx_hbm.at[pl.ds(t * 128, 128)], idx_v)
            for s in range(8):
                idx8_v[pl.ds(16 * s, 16)] = idx_v[pl.ds(16 * s, 16)] * 8
            set_gidx(0, 0)
            gather_start(0)
            set_gidx(1, 1)
            gather_start(1)

            def work(kk, p):
                gather_wait(p)

                @pl.when((kk >= 2) | (ti > 0))
                def _():
                    write_wait(kk, t, p)

                transpose(rows[p], blks[p], 128)

                @pl.when(kk + 2 <= KCH - 1)
                def _():
                    set_gidx(p, kk + 2)
                    gather_start(p)

                write_start(kk, t, p)

            def group(g, carry2):
                work(2 * g, 0)
                work(2 * g + 1, 1)
                return carry2

            lax.fori_loop(0, 3, group, 0)
            work(jnp.int32(6), 0)

            gather_wait(1)

            @pl.when(ti > 0)
            def _():
                pltpu.make_async_copy(
                    blk7.at[pl.ds(0, TAIL)],
                    out_hbm.at[pl.ds(896, TAIL), pl.ds(t * 128, 128)],
                    sem_w7,
                ).wait()

            transpose(rows[1], blk7, TAIL)
            pltpu.async_copy(
                blk7.at[pl.ds(0, TAIL)],
                out_hbm.at[pl.ds(896, TAIL), pl.ds(t * 128, 128)],
                sem_w7,
            )
            return carry

        lax.fori_loop(0, ntc, tile_col, 0)

        last_t = wid + (ntc - 1) * NW
        write_wait(jnp.int32(6), last_t, 0)
        write_wait(jnp.int32(5), last_t, 1)
        pltpu.make_async_copy(
            blk7.at[pl.ds(0, TAIL)],
            out_hbm.at[pl.ds(896, TAIL), pl.ds(last_t * 128, 128)],
            sem_w7,
        ).wait()

    return k(table_r, idx)


def kernel(X, table):
    idx = X.reshape(-1)
    table_r = jnp.pad(table, ((0, 0), (0, DPAD - D))).reshape(V * KCH, 128)
    out_t = _sc_gather_t(table_r, idx)
    return out_t.T


# speedup vs baseline: 1.6132x; 1.6132x over previous
import functools

import jax
import jax.numpy as jnp
from jax import lax
from jax.experimental import pallas as pl
from jax.experimental.pallas import tpu as pltpu
from jax.experimental.pallas import tpu_sc as plsc

V = 1000
D = 1000
DPAD = 1024
BULK = 896
TAIL = D - BULK
B = 1024 * 50
NC, NS = 2, 16
NW = NC * NS
B_PER_W = B // NW
CHUNK = 40
N_GROUPS = B_PER_W // (2 * CHUNK)


def _sc_gather(table, idx):
    mesh = plsc.VectorSubcoreMesh(core_axis_name="c", subcore_axis_name="s")

    @functools.partial(
        pl.kernel,
        mesh=mesh,
        out_type=jax.ShapeDtypeStruct((B, D), jnp.float32),
        scratch_types=[
            pltpu.VMEM((B_PER_W,), jnp.int32),
            pltpu.VMEM((CHUNK, DPAD), jnp.float32),
            pltpu.VMEM((CHUNK, DPAD), jnp.float32),
            pltpu.VMEM((CHUNK, TAIL), jnp.float32),
            pltpu.VMEM((CHUNK, TAIL), jnp.float32),
            pltpu.SemaphoreType.DMA,
            pltpu.SemaphoreType.DMA,
            pltpu.SemaphoreType.DMA,
            pltpu.SemaphoreType.DMA,
        ],
    )
    def k(table_hbm, idx_hbm, out_hbm, idx_v,
          rows0, rows1, tail0, tail1, sem_g0, sem_g1, sem_w0, sem_w1):
        cid = lax.axis_index("c")
        sid = lax.axis_index("s")
        wid = sid * NC + cid
        base = wid * B_PER_W
        pltpu.sync_copy(idx_hbm.at[pl.ds(base, B_PER_W)], idx_v)

        rows = (rows0, rows1)
        tails = (tail0, tail1)
        sem_g = (sem_g0, sem_g1)
        sem_w = (sem_w0, sem_w1)

        def gather_start(c, p):
            pltpu.async_copy(
                table_hbm.at[idx_v.at[pl.ds(c * CHUNK, CHUNK)]], rows[p], sem_g[p]
            )

        def gather_wait(c, p):
            pltpu.make_async_copy(
                table_hbm.at[idx_v.at[pl.ds(c * CHUNK, CHUNK)]], rows[p], sem_g[p]
            ).wait()

        def repack_tail(p):
            def row(i, carry):
                for t in range(6):
                    tails[p][i, pl.ds(t * 16, 16)] = rows[p][i, pl.ds(BULK + t * 16, 16)]
                tails[p][i, pl.ds(TAIL - 16, 16)] = rows[p][i, pl.ds(D - 16, 16)]
                return carry

            lax.fori_loop(0, CHUNK, row, 0)

        def write_start(c, p):
            o = base + c * CHUNK
            pltpu.async_copy(
                rows[p].at[:, pl.ds(0, BULK)],
                out_hbm.at[pl.ds(o, CHUNK), pl.ds(0, BULK)],
                sem_w[p],
            )
            pltpu.async_copy(
                tails[p], out_hbm.at[pl.ds(o, CHUNK), pl.ds(BULK, TAIL)], sem_w[p]
            )

        def write_wait(c, p):
            o = base + c * CHUNK
            pltpu.make_async_copy(
                rows[p].at[:, pl.ds(0, BULK)],
                out_hbm.at[pl.ds(o, CHUNK), pl.ds(0, BULK)],
                sem_w[p],
            ).wait()
            pltpu.make_async_copy(
                tails[p], out_hbm.at[pl.ds(o, CHUNK), pl.ds(BULK, TAIL)], sem_w[p]
            ).wait()

        gather_start(0, 0)

        def group(g, carry):
            c0 = 2 * g
            c1 = c0 + 1
            gather_wait(c0, 0)
            repack_tail(0)
            write_start(c0, 0)

            @pl.when(g > 0)
            def _():
                write_wait(c0 - 1, 1)

            gather_start(c1, 1)
            gather_wait(c1, 1)
            repack_tail(1)
            write_start(c1, 1)
            write_wait(c0, 0)

            @pl.when(g < N_GROUPS - 1)
            def _():
                gather_start(c0 + 2, 0)

            return carry

        lax.fori_loop(0, N_GROUPS, group, 0)
        write_wait(2 * N_GROUPS - 1, 1)

    return k(table, idx)


def kernel(X, table):
    idx = X.reshape(-1)
    table_pad = jnp.pad(table, ((0, 0), (0, DPAD - D)))
    return _sc_gather(table_pad, idx)


# speedup vs baseline: 2.7980x; 1.7345x over previous
import functools

import jax
import jax.numpy as jnp
from jax import lax
from jax.experimental import pallas as pl
from jax.experimental.pallas import tpu as pltpu
from jax.experimental.pallas import tpu_sc as plsc

V = 1000
D = 1000
DPAD = 1024
B = 1024 * 50
NC, NS = 2, 16
NW = NC * NS
NTC = B // 128
KCH = DPAD // 128
TAIL = D - 896


def _sc_gather_t(table_r, idx):
    mesh = plsc.VectorSubcoreMesh(core_axis_name="c", subcore_axis_name="s")

    @functools.partial(
        pl.kernel,
        mesh=mesh,
        compiler_params=pltpu.CompilerParams(needs_layout_passes=False),
        out_type=jax.ShapeDtypeStruct((D, B), jnp.float32),
        scratch_types=[
            pltpu.VMEM((128,), jnp.int32),
            pltpu.VMEM((128,), jnp.int32),
            pltpu.VMEM((128,), jnp.int32),
            pltpu.VMEM((128,), jnp.int32),
            pltpu.VMEM((128, 128), jnp.float32),
            pltpu.VMEM((128, 128), jnp.float32),
            pltpu.VMEM((128, 128), jnp.float32),
            pltpu.VMEM((128, 128), jnp.float32),
            pltpu.VMEM((128, 128), jnp.float32),
            pltpu.SemaphoreType.DMA,
            pltpu.SemaphoreType.DMA,
            pltpu.SemaphoreType.DMA,
            pltpu.SemaphoreType.DMA,
            pltpu.SemaphoreType.DMA,
        ],
    )
    def k(table_hbm, idx_hbm, out_hbm, idx_v, idx8_v, gidx0, gidx1,
          rows0, rows1, blk0, blk1, blk7,
          sem_g0, sem_g1, sem_w0, sem_w1, sem_w7):
        cid = lax.axis_index("c")
        sid = lax.axis_index("s")
        wid = sid * NC + cid
        ntc = jnp.where(wid < NTC % NW, NTC // NW + 1, NTC // NW)

        rows = (rows0, rows1)
        gidx = (gidx0, gidx1)
        sem_g = (sem_g0, sem_g1)
        blks = (blk0, blk1)
        sem_w = (sem_w0, sem_w1)

        iota16 = lax.broadcasted_iota(jnp.int32, (16,), 0)
        iotas = [iota16 + 16 * sg for sg in range(8)]

        def set_gidx(p, kk):
            for s in range(8):
                gidx[p][pl.ds(16 * s, 16)] = idx8_v[pl.ds(16 * s, 16)] + kk

        def gather_start(p):
            pltpu.async_copy(table_hbm.at[gidx[p]], rows[p], sem_g[p])

        def gather_wait(p):
            pltpu.make_async_copy(table_hbm.at[gidx[p]], rows[p], sem_g[p]).wait()

        def transpose(src, dst, limit):
            def body(j, carry):
                for dd in range(4):
                    d = j * 4 + dd
                    dvec = (iota16 + d) & 127
                    mask = dvec < limit if limit < 128 else None
                    vals = [
                        plsc.load_gather(src, [iotas[sg], dvec]) for sg in range(8)
                    ]
                    for sg in range(8):
                        plsc.store_scatter(
                            dst, [dvec, iotas[sg]], vals[sg], mask=mask
                        )
                return carry

            lax.fori_loop(0, 32, body, 0)

        def write_start(kk, t, p):
            pltpu.async_copy(
                blks[p],
                out_hbm.at[pl.ds(kk * 128, 128), pl.ds(t * 128, 128)],
                sem_w[p],
            )

        def write_wait(kk, t, p):
            pltpu.make_async_copy(
                blks[p],
                out_hbm.at[pl.ds(kk * 128, 128), pl.ds(t * 128, 128)],
                sem_w[p],
            ).wait()

        def tile_col(ti, carry):
            t = wid + ti * NW
            pltpu.sync_copy(idx_hbm.at[pl.ds(t * 128, 128)], idx_v)
            for s in range(8):
                idx8_v[pl.ds(16 * s, 16)] = idx_v[pl.ds(16 * s, 16)] * 8
            set_gidx(0, 0)
            gather_start(0)
            set_gidx(1, 1)
            gather_start(1)

            def work(kk, p):
                gather_wait(p)

                @pl.when((kk >= 2) | (ti > 0))
                def _():
                    write_wait(kk, t, p)

                transpose(rows[p], blks[p], 128)

                @pl.when(kk + 2 <= KCH - 1)
                def _():
                    set_gidx(p, kk + 2)
                    gather_start(p)

                write_start(kk, t, p)

            def group(g, carry2):
                work(2 * g, 0)
                work(2 * g + 1, 1)
                return carry2

            lax.fori_loop(0, 3, group, 0)
            work(jnp.int32(6), 0)

            gather_wait(1)

            @pl.when(ti > 0)
            def _():
                pltpu.make_async_copy(
                    blk7.at[pl.ds(0, TAIL)],
                    out_hbm.at[pl.ds(896, TAIL), pl.ds(t * 128, 128)],
                    sem_w7,
                ).wait()

            transpose(rows[1], blk7, TAIL)
            pltpu.async_copy(
                blk7.at[pl.ds(0, TAIL)],
                out_hbm.at[pl.ds(896, TAIL), pl.ds(t * 128, 128)],
                sem_w7,
            )
            return carry

        lax.fori_loop(0, ntc, tile_col, 0)

        last_t = wid + (ntc - 1) * NW
        write_wait(jnp.int32(6), last_t, 0)
        write_wait(jnp.int32(5), last_t, 1)
        pltpu.make_async_copy(
            blk7.at[pl.ds(0, TAIL)],
            out_hbm.at[pl.ds(896, TAIL), pl.ds(last_t * 128, 128)],
            sem_w7,
        ).wait()

    return k(table_r, idx)


def kernel(X, table):
    idx = X.reshape(-1)
    table_r = jnp.pad(table, ((0, 0), (0, DPAD - D))).reshape(V * KCH, 128)
    out_t = _sc_gather_t(table_r, idx)
    return out_t.T
